# baseline (speedup 1.0000x reference)
import jax
import jax.numpy as jnp
from jax import lax
from jax.experimental import pallas as pl
from jax.experimental.pallas import tpu as pltpu
from jax.experimental.pallas import tpu_sc as plsc

_INFO = plsc.get_sparse_core_info()
_NC, _NS, _L = _INFO.num_cores, _INFO.num_subcores, _INFO.num_lanes
_NW = _NC * _NS

_D = 128
_N = 1024 * 200
_PER_W = _N // _NW
_CG = 128
_C = 256
_NCHUNK = _PER_W // _C
_NIDX = _PER_W // _CG
_NBUF = 3
_PF = 1


def _body(x_hbm, idx3_hbm, pe_hbm, out_hbm,
          idxw, pe_sh, xa0, xa1, xa2, sx, sg, so):
    xa = (xa0, xa1, xa2)
    sid = lax.axis_index("s")
    wid = sid * _NC + lax.axis_index("c")
    base = wid * _PER_W

    @pl.when(sid == 0)
    def _():
        pltpu.sync_copy(pe_hbm, pe_sh)

    pltpu.sync_copy(idx3_hbm.at[wid], idxw)
    plsc.subcore_barrier()

    def fill(g, i):
        pltpu.async_copy(x_hbm.at[pl.ds(base + g * _C, _C)], xa[i], sx.at[i])

    def gadd(g, i):
        pltpu.async_copy(
            pe_sh.at[idxw.at[2 * g]], xa[i].at[pl.ds(0, _CG)],
            sg.at[i], add=True)
        pltpu.async_copy(
            pe_sh.at[idxw.at[2 * g + 1]], xa[i].at[pl.ds(_CG, _CG)],
            sg.at[i], add=True)

    def wait_sx(i):
        pltpu.make_async_copy(x_hbm.at[pl.ds(0, _C)], xa[i], sx.at[i]).wait()

    def wait_sg(i):
        half = xa[i].at[pl.ds(0, _CG)]
        pltpu.make_async_copy(pe_sh.at[idxw.at[0]], half, sg.at[i]).wait()
        pltpu.make_async_copy(pe_sh.at[idxw.at[0]], half, sg.at[i]).wait()

    def out(g, i):
        pltpu.async_copy(xa[i], out_hbm.at[pl.ds(base + g * _C, _C)],
                         so.at[i])

    def wait_so(i):
        pltpu.make_async_copy(xa[i], out_hbm.at[pl.ds(0, _C)], so.at[i]).wait()

    for g in range(_PF):
        fill(g, g)

    def rnd(r, carry):
        for i in range(_NBUF):
            g = r * _NBUF + i
            j = (i + _PF) % _NBUF
            p = (i + _NBUF - 1) % _NBUF

            @pl.when(g >= _NBUF - _PF)
            def _():
                wait_so(j)

            @pl.when(g + _PF < _NCHUNK)
            def _():
                fill(g + _PF, j)

            wait_sx(i)
            gadd(g, i)

            @pl.when(g >= 1)
            def _():
                wait_sg(p)
                out(g - 1, p)
        return carry

    nfull = (_NCHUNK // _NBUF) * _NBUF
    lax.fori_loop(0, _NCHUNK // _NBUF, rnd, 0)

    for g in range(nfull, _NCHUNK):
        i = g % _NBUF
        j = (i + _PF) % _NBUF
        p = (i + _NBUF - 1) % _NBUF
        wait_so(j)
        if g + _PF < _NCHUNK:
            fill(g + _PF, j)
        wait_sx(i)
        gadd(g, i)
        wait_sg(p)
        out(g - 1, p)

    sL = (_NCHUNK - 1) % _NBUF
    sM = (_NCHUNK - 2) % _NBUF
    wait_sg(sL)
    out(_NCHUNK - 1, sL)
    wait_so(sM)
    wait_so(sL)


@jax.jit
def _run(x2, idx3, pe):
    mesh = plsc.VectorSubcoreMesh(core_axis_name="c", subcore_axis_name="s")
    kfn = pl.kernel(
        _body,
        out_type=jax.ShapeDtypeStruct((_N, _D), jnp.float32),
        mesh=mesh,
        scratch_types=[
            pltpu.VMEM((_NIDX, _CG), jnp.int32),
            pltpu.VMEM_SHARED((256, _D), jnp.float32),
            pltpu.VMEM((_C, _D), jnp.float32),
            pltpu.VMEM((_C, _D), jnp.float32),
            pltpu.VMEM((_C, _D), jnp.float32),
            pltpu.SemaphoreType.DMA((_NBUF,)),
            pltpu.SemaphoreType.DMA((_NBUF,)),
            pltpu.SemaphoreType.DMA((_NBUF,)),
        ],
    )
    return kfn(x2, idx3, pe)


def kernel(x, frame_indices, pe):
    B, T, D = x.shape
    x2 = x.reshape(B * T, D)
    idx3 = frame_indices.reshape(_NW, _NIDX, _CG).astype(jnp.int32)
    out = _run(x2, idx3, pe)
    return out.reshape(B, T, D)


# speedup vs baseline: 1.0089x; 1.0089x over previous
import jax
import jax.numpy as jnp
from jax import lax
from jax.experimental import pallas as pl
from jax.experimental.pallas import tpu as pltpu
from jax.experimental.pallas import tpu_sc as plsc

_INFO = plsc.get_sparse_core_info()
_NC, _NS, _L = _INFO.num_cores, _INFO.num_subcores, _INFO.num_lanes
_NW = _NC * _NS

_D = 128
_N = 1024 * 200
_PER_W = _N // _NW
_C = 128
_NCHUNK = _PER_W // _C
_NBUF = 5
_PF = 2


def _body(x_hbm, idx3_hbm, pe_hbm, out_hbm,
          idxw, pe_sh, xa0, xa1, xa2, xa3, xa4, sx, sg, so):
    xa = (xa0, xa1, xa2, xa3, xa4)
    sid = lax.axis_index("s")
    wid = sid * _NC + lax.axis_index("c")
    base = wid * _PER_W

    @pl.when(sid == 0)
    def _():
        pltpu.sync_copy(pe_hbm, pe_sh)

    pltpu.sync_copy(idx3_hbm.at[wid], idxw)
    plsc.subcore_barrier()

    def buf(i):
        return xa[i]

    def fill(g, i):
        pltpu.async_copy(x_hbm.at[pl.ds(base + g * _C, _C)], buf(i), sx.at[i])

    for g in range(_PF):
        fill(g, g)

    def rnd(r, carry):
        for i in range(_NBUF):
            g = r * _NBUF + i
            j = (i + _PF) % _NBUF
            p = (i + _NBUF - 1) % _NBUF

            @pl.when(g >= _NBUF - _PF)
            def _():
                pltpu.make_async_copy(
                    buf(j), out_hbm.at[pl.ds(0, _C)], so.at[j]).wait()

            @pl.when(g + _PF < _NCHUNK)
            def _():
                fill(g + _PF, j)

            pltpu.make_async_copy(
                x_hbm.at[pl.ds(0, _C)], buf(i), sx.at[i]).wait()
            pltpu.async_copy(
                pe_sh.at[idxw.at[g]], buf(i), sg.at[i], add=True)

            @pl.when(g >= 1)
            def _():
                pltpu.make_async_copy(
                    pe_sh.at[idxw.at[0]], buf(p), sg.at[p]).wait()
                pltpu.async_copy(
                    buf(p), out_hbm.at[pl.ds(base + (g - 1) * _C, _C)],
                    so.at[p])
        return carry

    lax.fori_loop(0, _NCHUNK // _NBUF, rnd, 0)
    last = (_NCHUNK - 1) % _NBUF
    pltpu.make_async_copy(pe_sh.at[idxw.at[0]], buf(last), sg.at[last]).wait()
    pltpu.async_copy(
        buf(last), out_hbm.at[pl.ds(base + (_NCHUNK - 1) * _C, _C)],
        so.at[last])
    for k in range(_NBUF - _PF):
        b = (last - k) % _NBUF
        pltpu.make_async_copy(buf(b), out_hbm.at[pl.ds(0, _C)], so.at[b]).wait()


@jax.jit
def _run(x2, idx3, pe):
    mesh = plsc.VectorSubcoreMesh(core_axis_name="c", subcore_axis_name="s")
    kfn = pl.kernel(
        _body,
        out_type=jax.ShapeDtypeStruct((_N, _D), jnp.float32),
        mesh=mesh,
        scratch_types=[
            pltpu.VMEM((_NCHUNK, _C), jnp.int32),
            pltpu.VMEM_SHARED((256, _D), jnp.float32),
            pltpu.VMEM((_C, _D), jnp.float32),
            pltpu.VMEM((_C, _D), jnp.float32),
            pltpu.VMEM((_C, _D), jnp.float32),
            pltpu.VMEM((_C, _D), jnp.float32),
            pltpu.VMEM((_C, _D), jnp.float32),
            pltpu.SemaphoreType.DMA((_NBUF,)),
            pltpu.SemaphoreType.DMA((_NBUF,)),
            pltpu.SemaphoreType.DMA((_NBUF,)),
        ],
    )
    return kfn(x2, idx3, pe)


def kernel(x, frame_indices, pe):
    B, T, D = x.shape
    x2 = x.reshape(B * T, D)
    idx3 = frame_indices.reshape(_NW, _NCHUNK, _C).astype(jnp.int32)
    out = _run(x2, idx3, pe)
    return out.reshape(B, T, D)
